# baseline (speedup 1.0000x reference)
import functools

import jax
import jax.numpy as jnp
from jax.experimental import pallas as pl
from jax.experimental.pallas import tpu as pltpu

_K = 1.0
_EPS = 1e-7
_LN_EPS = 1e-6


def _sigmoid(x):
    return 1.0 / (1.0 + jnp.exp(-x))


def _layernorm(z):
    m = jnp.mean(z, axis=-1, keepdims=True)
    zc = z - m
    v = jnp.mean(zc * zc, axis=-1, keepdims=True)
    return zc * jax.lax.rsqrt(v + _LN_EPS)


def _mpnn_kernel(
    x_ref,
    adj_ref,
    e_ref,
    w1t_ref,
    w2t_ref,
    w3t_ref,
    b_edge_ref,
    w_sh_ref,
    b_sh_ref,
    w_sc_ref,
    b_sc_ref,
    w_adant_ref,
    b_adan_ref,
    wn1t_ref,
    wn2t_ref,
    b_node_ref,
    wp_ref,
    bp_ref,
    jones_ref,
    out_ref,
    X_s,
    xa_s,
    xb_s,
    S1_s,
    S2_s,
    t_s,
    cnt_s,
    u_s,
    *,
    TI, TJ, NI, NJ, N, D,
):
    i = pl.program_id(0)
    j = pl.program_id(1)

    @pl.when((i == 0) & (j == 0))
    def _init_nodes():
        xin = x_ref[...]
        t0 = jnp.clip(xin[:, 0:1], 1.0 + _EPS, None)
        num = jnp.log(t0 + jnp.sqrt(jnp.clip(t0 * t0 - 1.0, 1e-14, None)))
        coef = num / jnp.sqrt(jnp.clip(t0 * t0 - 1.0, 1e-14, None))
        lane = jax.lax.broadcasted_iota(jnp.int32, (N, D), 1)
        X = jnp.where(lane == 0, 0.0, coef * xin)
        X_s[...] = X
        xa = (
            jnp.dot(X, w1t_ref[...], preferred_element_type=jnp.float32)
            + b_edge_ref[...]
        )
        xa_s[...] = xa - jnp.mean(xa, axis=-1, keepdims=True)
        xb = jnp.dot(X, w2t_ref[...], preferred_element_type=jnp.float32)
        xb_s[...] = xb - jnp.mean(xb, axis=-1, keepdims=True)
        G = jax.lax.dot_general(
            X, X, (((1,), (1,)), ((), ())),
            preferred_element_type=jnp.float32,
        )
        arg = jnp.clip(-G, 1.0 + _EPS, None)
        dist = jnp.log(arg + jnp.sqrt(jnp.clip(arg * arg - 1.0, 1e-14, None)))
        dist = jnp.clip(dist, 1e-6, 100.0)
        u_s[...] = dist * _sigmoid(dist)

    @pl.when((i == 0) & (j == 0))
    def _zero_acc():
        S1_s[...] = jnp.zeros_like(S1_s)
        S2_s[...] = jnp.zeros_like(S2_s)
        t_s[...] = jnp.zeros_like(t_s)
        cnt_s[...] = jnp.zeros_like(cnt_s)

    u = u_s[pl.ds(i * TI, TI), pl.ds(j * TJ, TJ)]
    m = (adj_ref[...] != 0).astype(jnp.float32)
    mu = m * u

    e_blk = e_ref[...]
    z2 = jnp.dot(
        e_blk.reshape(TI * TJ, D).astype(jnp.bfloat16),
        w3t_ref[...].astype(jnp.bfloat16),
        preferred_element_type=jnp.float32,
    ).astype(jnp.bfloat16)
    xa_i = xa_s[pl.ds(i * TI, TI), :].astype(jnp.bfloat16)
    xb_j = xb_s[pl.ds(j * TJ, TJ), :].astype(jnp.bfloat16)
    zc = z2.reshape(TI, TJ, D) + xa_i[:, None, :] + xb_j[None, :, :]
    zsq = zc * zc
    vfull = jnp.dot(
        zsq.reshape(TI * TJ, D),
        jones_ref[...],
        preferred_element_type=jnp.float32,
    ).reshape(TI, TJ, D)
    rfull = jax.lax.rsqrt(vfull + _LN_EPS).astype(jnp.bfloat16)
    p = zc * rfull
    m3 = m.astype(jnp.bfloat16)[:, :, None]
    mu3 = mu.astype(jnp.bfloat16)[:, :, None]

    def _jsum(w3):
        TJq = TJ // 2
        s = (p[:, :TJq, :] * w3[:, :TJq, :]
             + p[:, TJq:, :] * w3[:, TJq:, :])
        TJq //= 2
        s = s[:, :TJq, :] + s[:, TJq:, :]
        TJq //= 2
        s = s[:, :TJq, :] + s[:, TJq:, :]
        return jnp.sum(s, axis=1, dtype=jnp.float32)

    ds_i = pl.ds(i * TI, TI)
    S1_s[ds_i, :] += _jsum(m3)
    S2_s[ds_i, :] += _jsum(mu3)
    t_s[ds_i, :] += jnp.broadcast_to(
        jnp.sum(mu, axis=1, keepdims=True), (TI, 128))
    cnt_s[ds_i, :] += jnp.broadcast_to(
        jnp.sum(m, axis=1, keepdims=True), (TI, 128))

    @pl.when((i == NI - 1) & (j == NJ - 1))
    def _node_stage():
        S1 = S1_s[...]
        S2 = S2_s[...]
        t = t_s[:, 0:1]
        cnt = cnt_s[:, 0:1]
        sums = (
            (1.0 + b_sc_ref[...]) * S1
            + w_sc_ref[...] * S2
            + t * w_sh_ref[...]
            + cnt * b_sh_ref[...]
        )
        agg = sums / jnp.maximum(cnt, 1.0)

        ag = agg * _sigmoid(agg)
        adan = (
            jnp.dot(ag, w_adant_ref[...], preferred_element_type=jnp.float32)
            + b_adan_ref[...]
        )
        shift_n = adan[:, 0:D]
        scale_n = adan[:, D:2 * D]
        gate_n = adan[:, 2 * D:3 * D]

        Xi_f = X_s[...]
        zn = (
            jnp.dot(Xi_f, wn1t_ref[...], preferred_element_type=jnp.float32)
            + jnp.dot(agg, wn2t_ref[...], preferred_element_type=jnp.float32)
            + b_node_ref[...]
        )
        nh = _layernorm(zn)
        node_out = Xi_f + gate_n * (nh * (1.0 + scale_n) + shift_n)

        xs = (
            jnp.dot(node_out, wp_ref[...], preferred_element_type=jnp.float32)
            + bp_ref[...]
        )
        xt = jnp.sqrt(jnp.sum(xs * xs, axis=-1, keepdims=True) + _K)
        lane = jax.lax.broadcasted_iota(jnp.int32, (N, D), 1)
        out_ref[...] = jnp.where(lane == 0, xt, xs)


def kernel(x, adj, e, W_edge, b_edge, W_node, b_node, W_hyp, b_hyp,
           W_ada, b_ada, W_adan, b_adan):
    B, N, D = x.shape
    assert B == 1, "kernel specialized for B=1"
    TI = 128
    TJ = 128
    NI = N // TI
    NJ = N // TJ

    x2 = x.reshape(N, D)
    adj2 = adj.reshape(N, N)
    e3 = e.reshape(N, N, D)

    w1t = W_edge[:, 0:D].T
    w2t = W_edge[:, D:2 * D].T
    w3t = W_edge[:, 2 * D:3 * D].T
    w3t = w3t - jnp.mean(w3t, axis=1, keepdims=True)
    b_edge2 = b_edge.reshape(1, D)
    w_sh = W_ada[0:D, 0].reshape(1, D)
    w_sc = W_ada[D:2 * D, 0].reshape(1, D)
    b_sh = b_ada[0:D].reshape(1, D)
    b_sc = b_ada[D:2 * D].reshape(1, D)
    w_adant = W_adan.T
    b_adan2 = b_adan.reshape(1, 3 * D)
    wn1t = W_node[:, 0:D].T
    wn2t = W_node[:, D:2 * D].T
    b_node2 = b_node.reshape(1, D)
    wp = jnp.concatenate([jnp.zeros((D, 1), W_hyp.dtype), W_hyp.T], axis=1)
    bp = jnp.concatenate([jnp.zeros((1,), b_hyp.dtype), b_hyp]).reshape(1, D)

    body = functools.partial(_mpnn_kernel, TI=TI, TJ=TJ, NI=NI, NJ=NJ, N=N, D=D)

    full = lambda shape: pl.BlockSpec(shape, lambda i, j: (0,) * len(shape))
    out = pl.pallas_call(
        body,
        grid=(NI, NJ),
        in_specs=[
            full((N, D)),
            pl.BlockSpec((TI, TJ), lambda i, j: (i, j)),
            pl.BlockSpec((TI, TJ, D), lambda i, j: (i, j, 0)),
            full((D, D)), full((D, D)), full((D, D)),
            full((1, D)),
            full((1, D)), full((1, D)), full((1, D)), full((1, D)),
            full((D, 3 * D)), full((1, 3 * D)),
            full((D, D)), full((D, D)), full((1, D)),
            full((D, D)), full((1, D)),
            full((D, D)),
        ],
        out_specs=pl.BlockSpec((N, D), lambda i, j: (0, 0)),
        out_shape=jax.ShapeDtypeStruct((N, D), jnp.float32),
        scratch_shapes=[
            pltpu.VMEM((N, D), jnp.float32),
            pltpu.VMEM((N, D), jnp.float32),
            pltpu.VMEM((N, D), jnp.float32),
            pltpu.VMEM((N, D), jnp.float32),
            pltpu.VMEM((N, D), jnp.float32),
            pltpu.VMEM((N, 128), jnp.float32),
            pltpu.VMEM((N, 128), jnp.float32),
            pltpu.VMEM((N, N), jnp.float32),
        ],
        compiler_params=pltpu.CompilerParams(
            dimension_semantics=("arbitrary", "arbitrary"),
        ),
    )(x2, adj2, e3, w1t, w2t, w3t, b_edge2, w_sh, b_sh, w_sc, b_sc,
      w_adant, b_adan2, wn1t, wn2t, b_node2, wp, bp,
      jnp.full((D, D), 1.0 / D, dtype=jnp.bfloat16))

    return out.reshape(B, N, D)


# speedup vs baseline: 1.0034x; 1.0034x over previous
import functools

import jax
import jax.numpy as jnp
from jax.experimental import pallas as pl
from jax.experimental.pallas import tpu as pltpu

_K = 1.0
_EPS = 1e-7
_LN_EPS = 1e-6


def _sigmoid(x):
    return 1.0 / (1.0 + jnp.exp(-x))


def _layernorm(z):
    m = jnp.mean(z, axis=-1, keepdims=True)
    zc = z - m
    v = jnp.mean(zc * zc, axis=-1, keepdims=True)
    return zc * jax.lax.rsqrt(v + _LN_EPS)


def _mpnn_kernel(
    x_ref,
    adj_ref,
    e_ref,
    w1t_ref,
    w2t_ref,
    w3t_ref,
    b_edge_ref,
    w_sh_ref,
    b_sh_ref,
    w_sc_ref,
    b_sc_ref,
    w_adant_ref,
    b_adan_ref,
    wn1t_ref,
    wn2t_ref,
    b_node_ref,
    wp_ref,
    bp_ref,
    jones_ref,
    out_ref,
    X_s,
    xa_s,
    xb_s,
    S1_s,
    S2_s,
    t_s,
    cnt_s,
    u_s,
    *,
    TI, TJ, NI, NJ, N, D,
):
    i = pl.program_id(0)
    j = pl.program_id(1)

    @pl.when((i == 0) & (j == 0))
    def _init_nodes():
        xin = x_ref[...]
        t0 = jnp.clip(xin[:, 0:1], 1.0 + _EPS, None)
        num = jnp.log(t0 + jnp.sqrt(jnp.clip(t0 * t0 - 1.0, 1e-14, None)))
        coef = num / jnp.sqrt(jnp.clip(t0 * t0 - 1.0, 1e-14, None))
        lane = jax.lax.broadcasted_iota(jnp.int32, (N, D), 1)
        X = jnp.where(lane == 0, 0.0, coef * xin)
        X_s[...] = X
        xa = (
            jnp.dot(X, w1t_ref[...], preferred_element_type=jnp.float32)
            + b_edge_ref[...]
        )
        xa_s[...] = xa - jnp.mean(xa, axis=-1, keepdims=True)
        xb = jnp.dot(X, w2t_ref[...], preferred_element_type=jnp.float32)
        xb_s[...] = xb - jnp.mean(xb, axis=-1, keepdims=True)
        G = jax.lax.dot_general(
            X, X, (((1,), (1,)), ((), ())),
            preferred_element_type=jnp.float32,
        )
        arg = jnp.clip(-G, 1.0 + _EPS, None)
        dist = jnp.log(arg + jnp.sqrt(jnp.clip(arg * arg - 1.0, 1e-14, None)))
        dist = jnp.clip(dist, 1e-6, 100.0)
        u_s[...] = dist * _sigmoid(dist)

    @pl.when((i == 0) & (j == 0))
    def _zero_acc():
        S1_s[...] = jnp.zeros_like(S1_s)
        S2_s[...] = jnp.zeros_like(S2_s)
        t_s[...] = jnp.zeros_like(t_s)
        cnt_s[...] = jnp.zeros_like(cnt_s)

    u = u_s[pl.ds(i * TI, TI), pl.ds(j * TJ, TJ)]
    m = (adj_ref[...] != 0).astype(jnp.float32)
    mu = m * u

    e_blk = e_ref[...]
    z2 = jnp.dot(
        e_blk.reshape(TI * TJ, D).astype(jnp.bfloat16),
        w3t_ref[...].astype(jnp.bfloat16),
        preferred_element_type=jnp.float32,
    ).astype(jnp.bfloat16)
    xa_i = xa_s[pl.ds(i * TI, TI), :].astype(jnp.bfloat16)
    xb_j = xb_s[pl.ds(j * TJ, TJ), :].astype(jnp.bfloat16)
    zc = z2.reshape(TI, TJ, D) + xa_i[:, None, :] + xb_j[None, :, :]
    zsq = zc * zc
    vfull = jnp.dot(
        zsq.reshape(TI * TJ, D),
        jones_ref[...],
        preferred_element_type=jnp.float32,
    ).reshape(TI, TJ, D)
    rfull = jax.lax.rsqrt(vfull + _LN_EPS).astype(jnp.bfloat16)
    p = zc * rfull
    m3 = m.astype(jnp.bfloat16)[:, :, None]
    mu3 = mu.astype(jnp.bfloat16)[:, :, None]

    def _jsum(w3):
        TJq = TJ // 2
        s = (p[:, :TJq, :] * w3[:, :TJq, :]
             + p[:, TJq:, :] * w3[:, TJq:, :])
        TJq //= 2
        s = s[:, :TJq, :] + s[:, TJq:, :]
        TJq //= 2
        s = s[:, :TJq, :] + s[:, TJq:, :]
        return jnp.sum(s, axis=1, dtype=jnp.float32)

    ds_i = pl.ds(i * TI, TI)
    S1_s[ds_i, :] += _jsum(m3)
    S2_s[ds_i, :] += _jsum(mu3)
    t_s[ds_i, :] += jnp.broadcast_to(
        jnp.sum(mu, axis=1, keepdims=True), (TI, 128))
    cnt_s[ds_i, :] += jnp.broadcast_to(
        jnp.sum(m, axis=1, keepdims=True), (TI, 128))

    @pl.when((i == NI - 1) & (j == NJ - 1))
    def _node_stage():
        S1 = S1_s[...]
        S2 = S2_s[...]
        t = t_s[:, 0:1]
        cnt = cnt_s[:, 0:1]
        sums = (
            (1.0 + b_sc_ref[...]) * S1
            + w_sc_ref[...] * S2
            + t * w_sh_ref[...]
            + cnt * b_sh_ref[...]
        )
        agg = sums / jnp.maximum(cnt, 1.0)

        ag = agg * _sigmoid(agg)
        adan = (
            jnp.dot(ag.astype(jnp.bfloat16),
                    w_adant_ref[...].astype(jnp.bfloat16),
                    preferred_element_type=jnp.float32)
            + b_adan_ref[...]
        )
        shift_n = adan[:, 0:D]
        scale_n = adan[:, D:2 * D]
        gate_n = adan[:, 2 * D:3 * D]

        Xi_f = X_s[...]
        zn = (
            jnp.dot(Xi_f.astype(jnp.bfloat16),
                    wn1t_ref[...].astype(jnp.bfloat16),
                    preferred_element_type=jnp.float32)
            + jnp.dot(agg.astype(jnp.bfloat16),
                      wn2t_ref[...].astype(jnp.bfloat16),
                      preferred_element_type=jnp.float32)
            + b_node_ref[...]
        )
        nh = _layernorm(zn)
        node_out = Xi_f + gate_n * (nh * (1.0 + scale_n) + shift_n)

        xs = (
            jnp.dot(node_out, wp_ref[...], preferred_element_type=jnp.float32)
            + bp_ref[...]
        )
        xt = jnp.sqrt(jnp.sum(xs * xs, axis=-1, keepdims=True) + _K)
        lane = jax.lax.broadcasted_iota(jnp.int32, (N, D), 1)
        out_ref[...] = jnp.where(lane == 0, xt, xs)


def kernel(x, adj, e, W_edge, b_edge, W_node, b_node, W_hyp, b_hyp,
           W_ada, b_ada, W_adan, b_adan):
    B, N, D = x.shape
    assert B == 1, "kernel specialized for B=1"
    TI = 128
    TJ = 128
    NI = N // TI
    NJ = N // TJ

    x2 = x.reshape(N, D)
    adj2 = adj.reshape(N, N)
    e3 = e.reshape(N, N, D)

    w1t = W_edge[:, 0:D].T
    w2t = W_edge[:, D:2 * D].T
    w3t = W_edge[:, 2 * D:3 * D].T
    w3t = w3t - jnp.mean(w3t, axis=1, keepdims=True)
    b_edge2 = b_edge.reshape(1, D)
    w_sh = W_ada[0:D, 0].reshape(1, D)
    w_sc = W_ada[D:2 * D, 0].reshape(1, D)
    b_sh = b_ada[0:D].reshape(1, D)
    b_sc = b_ada[D:2 * D].reshape(1, D)
    w_adant = W_adan.T
    b_adan2 = b_adan.reshape(1, 3 * D)
    wn1t = W_node[:, 0:D].T
    wn2t = W_node[:, D:2 * D].T
    b_node2 = b_node.reshape(1, D)
    wp = jnp.concatenate([jnp.zeros((D, 1), W_hyp.dtype), W_hyp.T], axis=1)
    bp = jnp.concatenate([jnp.zeros((1,), b_hyp.dtype), b_hyp]).reshape(1, D)

    body = functools.partial(_mpnn_kernel, TI=TI, TJ=TJ, NI=NI, NJ=NJ, N=N, D=D)

    full = lambda shape: pl.BlockSpec(shape, lambda i, j: (0,) * len(shape))
    out = pl.pallas_call(
        body,
        grid=(NI, NJ),
        in_specs=[
            full((N, D)),
            pl.BlockSpec((TI, TJ), lambda i, j: (i, j)),
            pl.BlockSpec((TI, TJ, D), lambda i, j: (i, j, 0)),
            full((D, D)), full((D, D)), full((D, D)),
            full((1, D)),
            full((1, D)), full((1, D)), full((1, D)), full((1, D)),
            full((D, 3 * D)), full((1, 3 * D)),
            full((D, D)), full((D, D)), full((1, D)),
            full((D, D)), full((1, D)),
            full((D, D)),
        ],
        out_specs=pl.BlockSpec((N, D), lambda i, j: (0, 0)),
        out_shape=jax.ShapeDtypeStruct((N, D), jnp.float32),
        scratch_shapes=[
            pltpu.VMEM((N, D), jnp.float32),
            pltpu.VMEM((N, D), jnp.float32),
            pltpu.VMEM((N, D), jnp.float32),
            pltpu.VMEM((N, D), jnp.float32),
            pltpu.VMEM((N, D), jnp.float32),
            pltpu.VMEM((N, 128), jnp.float32),
            pltpu.VMEM((N, 128), jnp.float32),
            pltpu.VMEM((N, N), jnp.float32),
        ],
        compiler_params=pltpu.CompilerParams(
            dimension_semantics=("arbitrary", "arbitrary"),
        ),
    )(x2, adj2, e3, w1t, w2t, w3t, b_edge2, w_sh, b_sh, w_sc, b_sc,
      w_adant, b_adan2, wn1t, wn2t, b_node2, wp, bp,
      jnp.full((D, D), 1.0 / D, dtype=jnp.bfloat16))

    return out.reshape(B, N, D)


# speedup vs baseline: 1.0181x; 1.0146x over previous
import functools

import jax
import jax.numpy as jnp
from jax.experimental import pallas as pl
from jax.experimental.pallas import tpu as pltpu

_K = 1.0
_EPS = 1e-7
_LN_EPS = 1e-6


def _sigmoid(x):
    return 1.0 / (1.0 + jnp.exp(-x))


def _layernorm(z):
    m = jnp.mean(z, axis=-1, keepdims=True)
    zc = z - m
    v = jnp.mean(zc * zc, axis=-1, keepdims=True)
    return zc * jax.lax.rsqrt(v + _LN_EPS)


def _mpnn_kernel(
    x_ref,
    adj_ref,
    e_ref,
    w1t_ref,
    w2t_ref,
    w3t_ref,
    b_edge_ref,
    w_sh_ref,
    b_sh_ref,
    w_sc_ref,
    b_sc_ref,
    w_adant_ref,
    b_adan_ref,
    wn1t_ref,
    wn2t_ref,
    b_node_ref,
    wp_ref,
    bp_ref,
    jones_ref,
    out_ref,
    X_s,
    xa_s,
    xb_s,
    S1_s,
    S2_s,
    t_s,
    cnt_s,
    u_s,
    *,
    TI, TJ, NI, NJ, N, D,
):
    i = pl.program_id(0)
    j = pl.program_id(1)

    @pl.when((i == 0) & (j == 0))
    def _init_nodes():
        xin = x_ref[...]
        t0 = jnp.clip(xin[:, 0:1], 1.0 + _EPS, None)
        num = jnp.log(t0 + jnp.sqrt(jnp.clip(t0 * t0 - 1.0, 1e-14, None)))
        coef = num / jnp.sqrt(jnp.clip(t0 * t0 - 1.0, 1e-14, None))
        lane = jax.lax.broadcasted_iota(jnp.int32, (N, D), 1)
        X = jnp.where(lane == 0, 0.0, coef * xin)
        X_s[...] = X
        xa = (
            jnp.dot(X, w1t_ref[...], preferred_element_type=jnp.float32)
            + b_edge_ref[...]
        )
        xa_s[...] = xa - jnp.mean(xa, axis=-1, keepdims=True)
        xb = jnp.dot(X, w2t_ref[...], preferred_element_type=jnp.float32)
        xb_s[...] = xb - jnp.mean(xb, axis=-1, keepdims=True)
        G = jax.lax.dot_general(
            X, X, (((1,), (1,)), ((), ())),
            preferred_element_type=jnp.float32,
        )
        arg = jnp.clip(-G, 1.0 + _EPS, None)
        dist = jnp.log(arg + jnp.sqrt(jnp.clip(arg * arg - 1.0, 1e-14, None)))
        dist = jnp.clip(dist, 1e-6, 100.0)
        u_s[...] = dist * _sigmoid(dist)

    @pl.when((i == 0) & (j == 0))
    def _zero_acc():
        S1_s[...] = jnp.zeros_like(S1_s)
        S2_s[...] = jnp.zeros_like(S2_s)
        t_s[...] = jnp.zeros_like(t_s)
        cnt_s[...] = jnp.zeros_like(cnt_s)

    u = u_s[pl.ds(i * TI, TI), pl.ds(j * TJ, TJ)]
    m = (adj_ref[...] != 0).astype(jnp.float32)
    mu = m * u

    e_blk = e_ref[...]
    z2 = jnp.dot(
        e_blk.reshape(TI * TJ, D).astype(jnp.bfloat16),
        w3t_ref[...].astype(jnp.bfloat16),
        preferred_element_type=jnp.float32,
    ).astype(jnp.bfloat16)
    xa_i = xa_s[pl.ds(i * TI, TI), :].astype(jnp.bfloat16)
    xb_j = xb_s[pl.ds(j * TJ, TJ), :].astype(jnp.bfloat16)
    zc = z2.reshape(TI, TJ, D) + xa_i[:, None, :] + xb_j[None, :, :]
    zsq = zc * zc
    vfull = jnp.dot(
        zsq.reshape(TI * TJ, D),
        jones_ref[...],
        preferred_element_type=jnp.float32,
    ).reshape(TI, TJ, D)
    rfull = jax.lax.rsqrt(vfull.astype(jnp.bfloat16)
                          + jnp.bfloat16(_LN_EPS))
    p = zc * rfull
    m3 = m.astype(jnp.bfloat16)[:, :, None]
    mu3 = mu.astype(jnp.bfloat16)[:, :, None]

    def _jsum(w3):
        TJq = TJ // 2
        s = (p[:, :TJq, :] * w3[:, :TJq, :]
             + p[:, TJq:, :] * w3[:, TJq:, :])
        TJq //= 2
        s = s[:, :TJq, :] + s[:, TJq:, :]
        TJq //= 2
        s = s[:, :TJq, :] + s[:, TJq:, :]
        return jnp.sum(s, axis=1, dtype=jnp.float32)

    ds_i = pl.ds(i * TI, TI)
    S1_s[ds_i, :] += _jsum(m3)
    S2_s[ds_i, :] += _jsum(mu3)
    t_s[ds_i, :] += jnp.broadcast_to(
        jnp.sum(mu, axis=1, keepdims=True), (TI, 128))
    cnt_s[ds_i, :] += jnp.broadcast_to(
        jnp.sum(m, axis=1, keepdims=True), (TI, 128))

    @pl.when((i == NI - 1) & (j == NJ - 1))
    def _node_stage():
        S1 = S1_s[...]
        S2 = S2_s[...]
        t = t_s[:, 0:1]
        cnt = cnt_s[:, 0:1]
        sums = (
            (1.0 + b_sc_ref[...]) * S1
            + w_sc_ref[...] * S2
            + t * w_sh_ref[...]
            + cnt * b_sh_ref[...]
        )
        agg = sums / jnp.maximum(cnt, 1.0)

        ag = agg * _sigmoid(agg)
        adan = (
            jnp.dot(ag.astype(jnp.bfloat16),
                    w_adant_ref[...].astype(jnp.bfloat16),
                    preferred_element_type=jnp.float32)
            + b_adan_ref[...]
        )
        shift_n = adan[:, 0:D]
        scale_n = adan[:, D:2 * D]
        gate_n = adan[:, 2 * D:3 * D]

        Xi_f = X_s[...]
        zn = (
            jnp.dot(Xi_f.astype(jnp.bfloat16),
                    wn1t_ref[...].astype(jnp.bfloat16),
                    preferred_element_type=jnp.float32)
            + jnp.dot(agg.astype(jnp.bfloat16),
                      wn2t_ref[...].astype(jnp.bfloat16),
                      preferred_element_type=jnp.float32)
            + b_node_ref[...]
        )
        nh = _layernorm(zn)
        node_out = Xi_f + gate_n * (nh * (1.0 + scale_n) + shift_n)

        xs = (
            jnp.dot(node_out, wp_ref[...], preferred_element_type=jnp.float32)
            + bp_ref[...]
        )
        xt = jnp.sqrt(jnp.sum(xs * xs, axis=-1, keepdims=True) + _K)
        lane = jax.lax.broadcasted_iota(jnp.int32, (N, D), 1)
        out_ref[...] = jnp.where(lane == 0, xt, xs)


def kernel(x, adj, e, W_edge, b_edge, W_node, b_node, W_hyp, b_hyp,
           W_ada, b_ada, W_adan, b_adan):
    B, N, D = x.shape
    assert B == 1, "kernel specialized for B=1"
    TI = 128
    TJ = 128
    NI = N // TI
    NJ = N // TJ

    x2 = x.reshape(N, D)
    adj2 = adj.reshape(N, N)
    e3 = e.reshape(N, N, D)

    w1t = W_edge[:, 0:D].T
    w2t = W_edge[:, D:2 * D].T
    w3t = W_edge[:, 2 * D:3 * D].T
    w3t = w3t - jnp.mean(w3t, axis=1, keepdims=True)
    b_edge2 = b_edge.reshape(1, D)
    w_sh = W_ada[0:D, 0].reshape(1, D)
    w_sc = W_ada[D:2 * D, 0].reshape(1, D)
    b_sh = b_ada[0:D].reshape(1, D)
    b_sc = b_ada[D:2 * D].reshape(1, D)
    w_adant = W_adan.T
    b_adan2 = b_adan.reshape(1, 3 * D)
    wn1t = W_node[:, 0:D].T
    wn2t = W_node[:, D:2 * D].T
    b_node2 = b_node.reshape(1, D)
    wp = jnp.concatenate([jnp.zeros((D, 1), W_hyp.dtype), W_hyp.T], axis=1)
    bp = jnp.concatenate([jnp.zeros((1,), b_hyp.dtype), b_hyp]).reshape(1, D)

    body = functools.partial(_mpnn_kernel, TI=TI, TJ=TJ, NI=NI, NJ=NJ, N=N, D=D)

    full = lambda shape: pl.BlockSpec(shape, lambda i, j: (0,) * len(shape))
    out = pl.pallas_call(
        body,
        grid=(NI, NJ),
        in_specs=[
            full((N, D)),
            pl.BlockSpec((TI, TJ), lambda i, j: (i, j)),
            pl.BlockSpec((TI, TJ, D), lambda i, j: (i, j, 0)),
            full((D, D)), full((D, D)), full((D, D)),
            full((1, D)),
            full((1, D)), full((1, D)), full((1, D)), full((1, D)),
            full((D, 3 * D)), full((1, 3 * D)),
            full((D, D)), full((D, D)), full((1, D)),
            full((D, D)), full((1, D)),
            full((D, D)),
        ],
        out_specs=pl.BlockSpec((N, D), lambda i, j: (0, 0)),
        out_shape=jax.ShapeDtypeStruct((N, D), jnp.float32),
        scratch_shapes=[
            pltpu.VMEM((N, D), jnp.float32),
            pltpu.VMEM((N, D), jnp.float32),
            pltpu.VMEM((N, D), jnp.float32),
            pltpu.VMEM((N, D), jnp.float32),
            pltpu.VMEM((N, D), jnp.float32),
            pltpu.VMEM((N, 128), jnp.float32),
            pltpu.VMEM((N, 128), jnp.float32),
            pltpu.VMEM((N, N), jnp.float32),
        ],
        compiler_params=pltpu.CompilerParams(
            dimension_semantics=("arbitrary", "arbitrary"),
        ),
    )(x2, adj2, e3, w1t, w2t, w3t, b_edge2, w_sh, b_sh, w_sc, b_sc,
      w_adant, b_adan2, wn1t, wn2t, b_node2, wp, bp,
      jnp.full((D, D), 1.0 / D, dtype=jnp.bfloat16))

    return out.reshape(B, N, D)


# speedup vs baseline: 1.0197x; 1.0016x over previous
import functools

import jax
import jax.numpy as jnp
from jax.experimental import pallas as pl
from jax.experimental.pallas import tpu as pltpu

_K = 1.0
_EPS = 1e-7
_LN_EPS = 1e-6


def _sigmoid(x):
    return 1.0 / (1.0 + jnp.exp(-x))


def _layernorm(z):
    m = jnp.mean(z, axis=-1, keepdims=True)
    zc = z - m
    v = jnp.mean(zc * zc, axis=-1, keepdims=True)
    return zc * jax.lax.rsqrt(v + _LN_EPS)


def _mpnn_kernel(
    x_ref,
    adj_ref,
    e_ref,
    w1t_ref,
    w2t_ref,
    w3t_ref,
    b_edge_ref,
    w_sh_ref,
    b_sh_ref,
    w_sc_ref,
    b_sc_ref,
    w_adant_ref,
    b_adan_ref,
    wn1t_ref,
    wn2t_ref,
    b_node_ref,
    wp_ref,
    bp_ref,
    jones_ref,
    out_ref,
    X_s,
    xa_s,
    xb_s,
    S1_s,
    S2_s,
    t_s,
    cnt_s,
    u_s,
    *,
    TI, TJ, NI, NJ, N, D,
):
    i = pl.program_id(0)
    j = pl.program_id(1)

    @pl.when((i == 0) & (j == 0))
    def _init_nodes():
        xin = x_ref[...]
        t0 = jnp.clip(xin[:, 0:1], 1.0 + _EPS, None)
        num = jnp.log(t0 + jnp.sqrt(jnp.clip(t0 * t0 - 1.0, 1e-14, None)))
        coef = num / jnp.sqrt(jnp.clip(t0 * t0 - 1.0, 1e-14, None))
        lane = jax.lax.broadcasted_iota(jnp.int32, (N, D), 1)
        X = jnp.where(lane == 0, 0.0, coef * xin)
        X_s[...] = X
        xa = (
            jnp.dot(X, w1t_ref[...], preferred_element_type=jnp.float32)
            + b_edge_ref[...]
        )
        xa_s[...] = xa - jnp.mean(xa, axis=-1, keepdims=True)
        xb = jnp.dot(X, w2t_ref[...], preferred_element_type=jnp.float32)
        xb_s[...] = xb - jnp.mean(xb, axis=-1, keepdims=True)
        G = jax.lax.dot_general(
            X, X, (((1,), (1,)), ((), ())),
            preferred_element_type=jnp.float32,
        )
        arg = jnp.clip(-G, 1.0 + _EPS, None)
        dist = jnp.log(arg + jnp.sqrt(jnp.clip(arg * arg - 1.0, 1e-14, None)))
        dist = jnp.clip(dist, 1e-6, 100.0)
        u_s[...] = dist * _sigmoid(dist)

    @pl.when((i == 0) & (j == 0))
    def _zero_acc():
        S1_s[...] = jnp.zeros_like(S1_s)
        S2_s[...] = jnp.zeros_like(S2_s)
        t_s[...] = jnp.zeros_like(t_s)
        cnt_s[...] = jnp.zeros_like(cnt_s)

    u = u_s[pl.ds(i * TI, TI), pl.ds(j * TJ, TJ)]
    m = (adj_ref[...] != 0).astype(jnp.float32)
    mu = m * u

    e_blk = e_ref[...]
    z2 = jnp.dot(
        e_blk.reshape(TI * TJ, D).astype(jnp.bfloat16),
        w3t_ref[...].astype(jnp.bfloat16),
        preferred_element_type=jnp.float32,
    ).astype(jnp.bfloat16)
    xa_i = xa_s[pl.ds(i * TI, TI), :].astype(jnp.bfloat16)
    xb_j = xb_s[pl.ds(j * TJ, TJ), :].astype(jnp.bfloat16)
    zc = z2.reshape(TI, TJ, D) + xa_i[:, None, :] + xb_j[None, :, :]
    zsq = zc * zc
    vfull = jnp.dot(
        zsq.reshape(TI * TJ, D),
        jones_ref[...],
        preferred_element_type=jnp.float32,
    ).reshape(TI, TJ, D)
    rfull = jax.lax.rsqrt(vfull.astype(jnp.bfloat16)
                          + jnp.bfloat16(_LN_EPS))
    p = zc * rfull
    m3 = m.astype(jnp.bfloat16)[:, :, None]
    mu3 = mu.astype(jnp.bfloat16)[:, :, None]

    def _jsum(w3):
        TJq = TJ // 2
        s = (p[:, :TJq, :] * w3[:, :TJq, :]
             + p[:, TJq:, :] * w3[:, TJq:, :])
        TJq //= 2
        s = s[:, :TJq, :] + s[:, TJq:, :]
        TJq //= 2
        s = s[:, :TJq, :] + s[:, TJq:, :]
        TJq //= 2
        s = s[:, :TJq, :] + s[:, TJq:, :]
        return jnp.sum(s, axis=1, dtype=jnp.float32)

    ds_i = pl.ds(i * TI, TI)
    S1_s[ds_i, :] += _jsum(m3)
    S2_s[ds_i, :] += _jsum(mu3)
    t_s[ds_i, :] += jnp.broadcast_to(
        jnp.sum(mu, axis=1, keepdims=True), (TI, 128))
    cnt_s[ds_i, :] += jnp.broadcast_to(
        jnp.sum(m, axis=1, keepdims=True), (TI, 128))

    @pl.when((i == NI - 1) & (j == NJ - 1))
    def _node_stage():
        S1 = S1_s[...]
        S2 = S2_s[...]
        t = t_s[:, 0:1]
        cnt = cnt_s[:, 0:1]
        sums = (
            (1.0 + b_sc_ref[...]) * S1
            + w_sc_ref[...] * S2
            + t * w_sh_ref[...]
            + cnt * b_sh_ref[...]
        )
        agg = sums / jnp.maximum(cnt, 1.0)

        ag = agg * _sigmoid(agg)
        adan = (
            jnp.dot(ag.astype(jnp.bfloat16),
                    w_adant_ref[...].astype(jnp.bfloat16),
                    preferred_element_type=jnp.float32)
            + b_adan_ref[...]
        )
        shift_n = adan[:, 0:D]
        scale_n = adan[:, D:2 * D]
        gate_n = adan[:, 2 * D:3 * D]

        Xi_f = X_s[...]
        zn = (
            jnp.dot(Xi_f.astype(jnp.bfloat16),
                    wn1t_ref[...].astype(jnp.bfloat16),
                    preferred_element_type=jnp.float32)
            + jnp.dot(agg.astype(jnp.bfloat16),
                      wn2t_ref[...].astype(jnp.bfloat16),
                      preferred_element_type=jnp.float32)
            + b_node_ref[...]
        )
        nh = _layernorm(zn)
        node_out = Xi_f + gate_n * (nh * (1.0 + scale_n) + shift_n)

        xs = (
            jnp.dot(node_out, wp_ref[...], preferred_element_type=jnp.float32)
            + bp_ref[...]
        )
        xt = jnp.sqrt(jnp.sum(xs * xs, axis=-1, keepdims=True) + _K)
        lane = jax.lax.broadcasted_iota(jnp.int32, (N, D), 1)
        out_ref[...] = jnp.where(lane == 0, xt, xs)


def kernel(x, adj, e, W_edge, b_edge, W_node, b_node, W_hyp, b_hyp,
           W_ada, b_ada, W_adan, b_adan):
    B, N, D = x.shape
    assert B == 1, "kernel specialized for B=1"
    TI = 128
    TJ = 128
    NI = N // TI
    NJ = N // TJ

    x2 = x.reshape(N, D)
    adj2 = adj.reshape(N, N)
    e3 = e.reshape(N, N, D)

    w1t = W_edge[:, 0:D].T
    w2t = W_edge[:, D:2 * D].T
    w3t = W_edge[:, 2 * D:3 * D].T
    w3t = w3t - jnp.mean(w3t, axis=1, keepdims=True)
    b_edge2 = b_edge.reshape(1, D)
    w_sh = W_ada[0:D, 0].reshape(1, D)
    w_sc = W_ada[D:2 * D, 0].reshape(1, D)
    b_sh = b_ada[0:D].reshape(1, D)
    b_sc = b_ada[D:2 * D].reshape(1, D)
    w_adant = W_adan.T
    b_adan2 = b_adan.reshape(1, 3 * D)
    wn1t = W_node[:, 0:D].T
    wn2t = W_node[:, D:2 * D].T
    b_node2 = b_node.reshape(1, D)
    wp = jnp.concatenate([jnp.zeros((D, 1), W_hyp.dtype), W_hyp.T], axis=1)
    bp = jnp.concatenate([jnp.zeros((1,), b_hyp.dtype), b_hyp]).reshape(1, D)

    body = functools.partial(_mpnn_kernel, TI=TI, TJ=TJ, NI=NI, NJ=NJ, N=N, D=D)

    full = lambda shape: pl.BlockSpec(shape, lambda i, j: (0,) * len(shape))
    out = pl.pallas_call(
        body,
        grid=(NI, NJ),
        in_specs=[
            full((N, D)),
            pl.BlockSpec((TI, TJ), lambda i, j: (i, j)),
            pl.BlockSpec((TI, TJ, D), lambda i, j: (i, j, 0)),
            full((D, D)), full((D, D)), full((D, D)),
            full((1, D)),
            full((1, D)), full((1, D)), full((1, D)), full((1, D)),
            full((D, 3 * D)), full((1, 3 * D)),
            full((D, D)), full((D, D)), full((1, D)),
            full((D, D)), full((1, D)),
            full((D, D)),
        ],
        out_specs=pl.BlockSpec((N, D), lambda i, j: (0, 0)),
        out_shape=jax.ShapeDtypeStruct((N, D), jnp.float32),
        scratch_shapes=[
            pltpu.VMEM((N, D), jnp.float32),
            pltpu.VMEM((N, D), jnp.float32),
            pltpu.VMEM((N, D), jnp.float32),
            pltpu.VMEM((N, D), jnp.float32),
            pltpu.VMEM((N, D), jnp.float32),
            pltpu.VMEM((N, 128), jnp.float32),
            pltpu.VMEM((N, 128), jnp.float32),
            pltpu.VMEM((N, N), jnp.float32),
        ],
        compiler_params=pltpu.CompilerParams(
            dimension_semantics=("arbitrary", "arbitrary"),
        ),
    )(x2, adj2, e3, w1t, w2t, w3t, b_edge2, w_sh, b_sh, w_sc, b_sc,
      w_adant, b_adan2, wn1t, wn2t, b_node2, wp, bp,
      jnp.full((D, D), 1.0 / D, dtype=jnp.bfloat16))

    return out.reshape(B, N, D)
